# baseline (speedup 1.0000x reference)
import functools

import jax
import jax.numpy as jnp
from jax import lax
from jax.experimental import pallas as pl
from jax.experimental.pallas import tpu as pltpu
from jax.experimental.pallas import tpu_sc as plsc

C, R, T = 64, 512, 512
N = 2_000_000
L = 16
NV = N // L
NW = 32
NT = 2 * NW
VPT = -(-NV // NW)
CV = 384
CVL = CV * L
CPT = -(-VPT // CV)
U = 16
CT = C * T
CTP = CT + 128
DUMP = CT


def _segment_max(rho, theta, f):
    mesh = plsc.VectorSubcoreMesh(core_axis_name="c", subcore_axis_name="s")

    @functools.partial(
        pl.kernel,
        out_type=jax.ShapeDtypeStruct((NT, CT), jnp.int32),
        mesh=mesh,
        compiler_params=pltpu.CompilerParams(needs_layout_passes=False),
        scratch_types=[
            pltpu.VMEM((CVL,), jnp.int32),
            pltpu.VMEM((CVL,), jnp.int32),
            pltpu.VMEM((CVL,), jnp.int32),
            pltpu.VMEM((CVL,), jnp.int32),
            pltpu.VMEM((CVL,), jnp.int32),
            pltpu.VMEM((CVL,), jnp.int32),
            pltpu.VMEM((CVL,), jnp.int32),
            pltpu.VMEM((CVL,), jnp.int32),
            pltpu.VMEM((CTP,), jnp.int32),
            pltpu.VMEM((CTP,), jnp.int32),
            pltpu.SemaphoreType.DMA,
            pltpu.SemaphoreType.DMA,
        ],
    )
    def k(rho_h, th_h, f_h, out_h, rho_v0, rho_v1, th_v0, th_v1, f_v0, f_v1,
          bs_v, vs_v, acc_a, acc_b, s0, s1):
        wid = lax.axis_index("s") * 2 + lax.axis_index("c")
        tstart = jnp.minimum(wid * VPT, NV - VPT)
        sems = (s0, s1)
        rho_vs = (rho_v0, rho_v1)
        th_vs = (th_v0, th_v1)
        f_vs = (f_v0, f_v1)

        def start(ci, slot):
            cstart = jnp.minimum(tstart + ci * CV, tstart + (VPT - CV))
            p0 = cstart * L
            sem = sems[slot]
            return [
                pltpu.async_copy(rho_h.at[pl.ds(p0, CVL)], rho_vs[slot], sem),
                pltpu.async_copy(th_h.at[pl.ds(p0, CVL)], th_vs[slot], sem),
                pltpu.async_copy(f_h.at[pl.ds(p0, CVL)], f_vs[slot], sem),
            ]

        pending = {0: start(0, 0)}

        neg1 = jnp.full((L,), -1, jnp.int32)

        def initb(i, _):
            for u in range(8):
                off = (i * 8 + u) * L
                acc_a[pl.ds(off, L)] = neg1
                acc_b[pl.ds(off, L)] = neg1
            return 0

        lax.fori_loop(0, CTP // (8 * L), initb, 0)

        lane = lax.iota(jnp.int32, L)
        lanep1 = jnp.minimum(lane + 1, L - 1)
        top1 = (lane == L - 1).astype(jnp.int32)

        for ci in range(CPT):
            slot = ci % 2
            if ci + 1 < CPT:
                pending[ci + 1] = start(ci + 1, 1 - slot)
            for h in pending.pop(ci):
                h.wait()

            @plsc.parallel_loop(0, CV, unroll=8)
            def prep(j, slot=slot):
                base = j * L
                rho16 = rho_vs[slot][pl.ds(base, L)]
                th16 = th_vs[slot][pl.ds(base, L)]
                f16 = f_vs[slot][pl.ds(base, L)]
                key = ((f16 * T + th16) * 512 + rho16).astype(jnp.uint32)
                skey, srho = plsc.sort_key_val(key, rho16)
                b = lax.shift_right_logical(
                    skey, jnp.uint32(9)).astype(jnp.int32)
                nxt = jnp.take_along_axis(
                    b, lanep1, axis=0, mode="promise_in_bounds"
                ) + top1
                bs_v[pl.ds(base, L)] = jnp.where(b != nxt, b, DUMP)
                vs_v[pl.ds(base, L)] = srho

            def body(j, _):
                bs = []
                vs = []
                for u in range(U):
                    base = (j * U + u) * L
                    bs.append(bs_v[pl.ds(base, L)])
                    vs.append(vs_v[pl.ds(base, L)])
                for u in range(0, U, 2):
                    ca = plsc.load_gather(acc_a, [bs[u]])
                    cb = plsc.load_gather(acc_b, [bs[u + 1]])
                    plsc.store_scatter(acc_a, [bs[u]], jnp.maximum(ca, vs[u]))
                    plsc.store_scatter(
                        acc_b, [bs[u + 1]], jnp.maximum(cb, vs[u + 1]))
                return 0

            lax.fori_loop(0, CV // U, body, 0)

        pltpu.sync_copy(acc_a.at[pl.ds(0, CT)], out_h.at[2 * wid])
        pltpu.sync_copy(acc_b.at[pl.ds(0, CT)], out_h.at[2 * wid + 1])

    return k(rho, theta, f)


CB = 8


def _expand(partial):
    def body(p_ref, o_ref):
        p = p_ref[...]
        for cc in range(CB):
            sub = p[:, cc * T:(cc + 1) * T]
            m = jnp.max(sub, axis=0)
            m = jnp.where(m < 0, R, m)
            ii = lax.broadcasted_iota(jnp.int32, (R, T), 0)
            o_ref[cc] = (m[None, :] - ii).astype(jnp.float32)

    return pl.pallas_call(
        body,
        grid=(C // CB,),
        in_specs=[pl.BlockSpec((NT, CB * T), lambda cb: (0, cb))],
        out_specs=pl.BlockSpec((CB, R, T), lambda cb: (cb, 0, 0)),
        out_shape=jax.ShapeDtypeStruct((C, R, T), jnp.float32),
    )(partial)


def kernel(rho, theta, f, rp, r):
    part = _segment_max(rho, theta, f)
    return _expand(part)


# speedup vs baseline: 1.0589x; 1.0589x over previous
import functools

import jax
import jax.numpy as jnp
from jax import lax
from jax.experimental import pallas as pl
from jax.experimental.pallas import tpu as pltpu
from jax.experimental.pallas import tpu_sc as plsc

C, R, T = 64, 512, 512
N = 2_000_000
L = 16
NV = N // L
NW = 32
NT = 2 * NW
VPT = -(-NV // NW)
CV = 448
CVL = CV * L
CPT = -(-VPT // CV)
U = 8
CT = C * T
CTP = CT + 128
DUMP = CT


def _segment_max(rho, theta, f):
    mesh = plsc.VectorSubcoreMesh(core_axis_name="c", subcore_axis_name="s")

    @functools.partial(
        pl.kernel,
        out_type=jax.ShapeDtypeStruct((NT, CT), jnp.int32),
        mesh=mesh,
        compiler_params=pltpu.CompilerParams(needs_layout_passes=False),
        scratch_types=[
            pltpu.VMEM((CVL,), jnp.int32),
            pltpu.VMEM((CVL,), jnp.int32),
            pltpu.VMEM((CVL,), jnp.int32),
            pltpu.VMEM((CVL,), jnp.int32),
            pltpu.VMEM((CVL,), jnp.int32),
            pltpu.VMEM((CVL,), jnp.int32),
            pltpu.VMEM((CVL,), jnp.int32),
            pltpu.VMEM((CVL,), jnp.int32),
            pltpu.VMEM((CTP,), jnp.int32),
            pltpu.VMEM((CTP,), jnp.int32),
            pltpu.SemaphoreType.DMA,
            pltpu.SemaphoreType.DMA,
        ],
    )
    def k(rho_h, th_h, f_h, out_h, rho_v0, rho_v1, th_v0, th_v1, f_v0, f_v1,
          bs_v, vs_v, acc_a, acc_b, s0, s1):
        wid = lax.axis_index("s") * 2 + lax.axis_index("c")
        tstart = jnp.minimum(wid * VPT, NV - VPT)
        sems = (s0, s1)
        rho_vs = (rho_v0, rho_v1)
        th_vs = (th_v0, th_v1)
        f_vs = (f_v0, f_v1)

        def start(ci, slot):
            cstart = jnp.minimum(tstart + ci * CV, tstart + (VPT - CV))
            p0 = cstart * L
            sem = sems[slot]
            return [
                pltpu.async_copy(rho_h.at[pl.ds(p0, CVL)], rho_vs[slot], sem),
                pltpu.async_copy(th_h.at[pl.ds(p0, CVL)], th_vs[slot], sem),
                pltpu.async_copy(f_h.at[pl.ds(p0, CVL)], f_vs[slot], sem),
            ]

        pending = {0: start(0, 0)}

        neg1 = jnp.full((L,), -1, jnp.int32)

        def initb(i, _):
            for u in range(8):
                off = (i * 8 + u) * L
                acc_a[pl.ds(off, L)] = neg1
                acc_b[pl.ds(off, L)] = neg1
            return 0

        lax.fori_loop(0, CTP // (8 * L), initb, 0)

        lane = lax.iota(jnp.int32, L)
        lanep1 = jnp.minimum(lane + 1, L - 1)
        top1 = (lane == L - 1).astype(jnp.int32)

        for ci in range(CPT):
            slot = ci % 2
            if ci + 1 < CPT:
                pending[ci + 1] = start(ci + 1, 1 - slot)
            for h in pending.pop(ci):
                h.wait()

            @plsc.parallel_loop(0, CV, unroll=8)
            def prep(j, slot=slot):
                base = j * L
                rho16 = rho_vs[slot][pl.ds(base, L)]
                th16 = th_vs[slot][pl.ds(base, L)]
                f16 = f_vs[slot][pl.ds(base, L)]
                key = ((f16 * T + th16) * 512 + rho16).astype(jnp.uint32)
                skey, srho = plsc.sort_key_val(key, rho16)
                b = lax.shift_right_logical(
                    skey, jnp.uint32(9)).astype(jnp.int32)
                nxt = jnp.take_along_axis(
                    b, lanep1, axis=0, mode="promise_in_bounds"
                ) + top1
                bs_v[pl.ds(base, L)] = jnp.where(b != nxt, b, DUMP)
                vs_v[pl.ds(base, L)] = srho

            def body(j, _):
                bs = []
                vs = []
                for u in range(U):
                    base = (j * U + u) * L
                    bs.append(bs_v[pl.ds(base, L)])
                    vs.append(vs_v[pl.ds(base, L)])
                for u in range(0, U, 2):
                    ca = plsc.load_gather(acc_a, [bs[u]])
                    cb = plsc.load_gather(acc_b, [bs[u + 1]])
                    plsc.store_scatter(acc_a, [bs[u]], jnp.maximum(ca, vs[u]))
                    plsc.store_scatter(
                        acc_b, [bs[u + 1]], jnp.maximum(cb, vs[u + 1]))
                return 0

            lax.fori_loop(0, CV // U, body, 0)

        pltpu.sync_copy(acc_a.at[pl.ds(0, CT)], out_h.at[2 * wid])
        pltpu.sync_copy(acc_b.at[pl.ds(0, CT)], out_h.at[2 * wid + 1])

    return k(rho, theta, f)


CB = 8


def _expand(partial):
    def body(p_ref, o_ref):
        p = p_ref[...]
        for cc in range(CB):
            sub = p[:, cc * T:(cc + 1) * T]
            m = jnp.max(sub, axis=0)
            m = jnp.where(m < 0, R, m)
            ii = lax.broadcasted_iota(jnp.int32, (R, T), 0)
            o_ref[cc] = (m[None, :] - ii).astype(jnp.float32)

    return pl.pallas_call(
        body,
        grid=(C // CB,),
        in_specs=[pl.BlockSpec((NT, CB * T), lambda cb: (0, cb))],
        out_specs=pl.BlockSpec((CB, R, T), lambda cb: (cb, 0, 0)),
        out_shape=jax.ShapeDtypeStruct((C, R, T), jnp.float32),
    )(partial)


def kernel(rho, theta, f, rp, r):
    part = _segment_max(rho, theta, f)
    return _expand(part)


# speedup vs baseline: 1.0642x; 1.0050x over previous
import functools

import jax
import jax.numpy as jnp
from jax import lax
from jax.experimental import pallas as pl
from jax.experimental.pallas import tpu as pltpu
from jax.experimental.pallas import tpu_sc as plsc

C, R, T = 64, 512, 512
N = 2_000_000
L = 16
NV = N // L
NW = 32
NT = 2 * NW
VPT = -(-NV // NW)
CV = 440
CVL = CV * L
CPT = -(-VPT // CV)
U = 8
CT = C * T
CTP = CT + 128
DUMP = CT


def _segment_max(rho, theta, f):
    mesh = plsc.VectorSubcoreMesh(core_axis_name="c", subcore_axis_name="s")

    @functools.partial(
        pl.kernel,
        out_type=jax.ShapeDtypeStruct((NT, CT), jnp.int32),
        mesh=mesh,
        compiler_params=pltpu.CompilerParams(needs_layout_passes=False),
        scratch_types=[
            pltpu.VMEM((CVL,), jnp.int32),
            pltpu.VMEM((CVL,), jnp.int32),
            pltpu.VMEM((CVL,), jnp.int32),
            pltpu.VMEM((CVL,), jnp.int32),
            pltpu.VMEM((CVL,), jnp.int32),
            pltpu.VMEM((CVL,), jnp.int32),
            pltpu.VMEM((CVL,), jnp.int32),
            pltpu.VMEM((CVL,), jnp.int32),
            pltpu.VMEM((CTP,), jnp.int32),
            pltpu.VMEM((CTP,), jnp.int32),
            pltpu.SemaphoreType.DMA,
            pltpu.SemaphoreType.DMA,
        ],
    )
    def k(rho_h, th_h, f_h, out_h, rho_v0, rho_v1, th_v0, th_v1, f_v0, f_v1,
          bs_v, vs_v, acc_a, acc_b, s0, s1):
        wid = lax.axis_index("s") * 2 + lax.axis_index("c")
        tstart = jnp.minimum(wid * VPT, NV - VPT)
        sems = (s0, s1)
        rho_vs = (rho_v0, rho_v1)
        th_vs = (th_v0, th_v1)
        f_vs = (f_v0, f_v1)

        def start(ci, slot):
            cstart = jnp.minimum(tstart + ci * CV, tstart + (VPT - CV))
            p0 = cstart * L
            sem = sems[slot]
            return [
                pltpu.async_copy(rho_h.at[pl.ds(p0, CVL)], rho_vs[slot], sem),
                pltpu.async_copy(th_h.at[pl.ds(p0, CVL)], th_vs[slot], sem),
                pltpu.async_copy(f_h.at[pl.ds(p0, CVL)], f_vs[slot], sem),
            ]

        pending = {0: start(0, 0)}

        neg1 = jnp.full((L,), -1, jnp.int32)

        def initb(i, _):
            for u in range(8):
                off = (i * 8 + u) * L
                acc_a[pl.ds(off, L)] = neg1
                acc_b[pl.ds(off, L)] = neg1
            return 0

        lax.fori_loop(0, CTP // (8 * L), initb, 0)

        lane = lax.iota(jnp.int32, L)
        lanep1 = jnp.minimum(lane + 1, L - 1)
        top1 = (lane == L - 1).astype(jnp.int32)

        for ci in range(CPT):
            slot = ci % 2
            if ci + 1 < CPT:
                pending[ci + 1] = start(ci + 1, 1 - slot)
            for h in pending.pop(ci):
                h.wait()

            @plsc.parallel_loop(0, CV, unroll=8)
            def prep(j, slot=slot):
                base = j * L
                rho16 = rho_vs[slot][pl.ds(base, L)]
                th16 = th_vs[slot][pl.ds(base, L)]
                f16 = f_vs[slot][pl.ds(base, L)]
                key = ((f16 * T + th16) * 512 + rho16).astype(jnp.uint32)
                skey, srho = plsc.sort_key_val(key, rho16)
                b = lax.shift_right_logical(
                    skey, jnp.uint32(9)).astype(jnp.int32)
                nxt = jnp.take_along_axis(
                    b, lanep1, axis=0, mode="promise_in_bounds"
                ) + top1
                bs_v[pl.ds(base, L)] = jnp.where(b != nxt, b, DUMP)
                vs_v[pl.ds(base, L)] = srho

            def body(j, _):
                bs = []
                vs = []
                for u in range(U):
                    base = (j * U + u) * L
                    bs.append(bs_v[pl.ds(base, L)])
                    vs.append(vs_v[pl.ds(base, L)])
                for u in range(0, U, 2):
                    ca = plsc.load_gather(acc_a, [bs[u]])
                    cb = plsc.load_gather(acc_b, [bs[u + 1]])
                    plsc.store_scatter(acc_a, [bs[u]], jnp.maximum(ca, vs[u]))
                    plsc.store_scatter(
                        acc_b, [bs[u + 1]], jnp.maximum(cb, vs[u + 1]))
                return 0

            lax.fori_loop(0, CV // U, body, 0)

        pltpu.sync_copy(acc_a.at[pl.ds(0, CT)], out_h.at[2 * wid])
        pltpu.sync_copy(acc_b.at[pl.ds(0, CT)], out_h.at[2 * wid + 1])

    return k(rho, theta, f)


CB = 8


def _expand(partial):
    def body(p_ref, o_ref):
        p = p_ref[...]
        for cc in range(CB):
            sub = p[:, cc * T:(cc + 1) * T]
            m = jnp.max(sub, axis=0)
            m = jnp.where(m < 0, R, m)
            ii = lax.broadcasted_iota(jnp.int32, (R, T), 0)
            o_ref[cc] = (m[None, :] - ii).astype(jnp.float32)

    return pl.pallas_call(
        body,
        grid=(C // CB,),
        in_specs=[pl.BlockSpec((NT, CB * T), lambda cb: (0, cb))],
        out_specs=pl.BlockSpec((CB, R, T), lambda cb: (cb, 0, 0)),
        out_shape=jax.ShapeDtypeStruct((C, R, T), jnp.float32),
    )(partial)


def kernel(rho, theta, f, rp, r):
    part = _segment_max(rho, theta, f)
    return _expand(part)
